# baseline (speedup 1.0000x reference)
import functools

import jax
import jax.numpy as jnp
from jax import lax
from jax.experimental import pallas as pl
from jax.experimental.pallas import tpu as pltpu
from jax.experimental.pallas import tpu_sc as plsc

LN_EPS = 1e-5

_info = plsc.get_sparse_core_info()
_NC, _NS = _info.num_cores, _info.num_subcores
_NW = _NC * _NS


def _sc_gather(ids_flat, table, chunk):
    n = ids_flat.shape[0]
    d = table.shape[1]
    per_w = n // _NW
    n_chunks = per_w // chunk
    mesh = plsc.VectorSubcoreMesh(core_axis_name="c", subcore_axis_name="s")

    @functools.partial(
        pl.kernel,
        mesh=mesh,
        out_type=jax.ShapeDtypeStruct((n, d), jnp.float32),
        scratch_types=[
            pltpu.VMEM((per_w,), jnp.int32),
            pltpu.VMEM((2, chunk, d), jnp.float32),
            pltpu.SemaphoreType.DMA,
            pltpu.SemaphoreType.DMA,
        ],
    )
    def k(ids_hbm, table_hbm, out_hbm, idx_v, rows_v, gsem, osem):
        wid = lax.axis_index("s") * _NC + lax.axis_index("c")
        base = wid * per_w
        pltpu.sync_copy(ids_hbm.at[pl.ds(base, per_w)], idx_v)

        def issue(slot, ci):
            return pltpu.async_copy(
                table_hbm.at[idx_v.at[pl.ds(ci * chunk, chunk)]],
                rows_v.at[slot], gsem)

        g = issue(0, 0)
        wb = [None, None]
        for ci in range(n_chunks):
            slot = ci % 2
            g.wait()
            off = base + ci * chunk
            wb[slot] = pltpu.async_copy(rows_v.at[slot],
                                        out_hbm.at[pl.ds(off, chunk)], osem)
            if ci + 1 < n_chunks:
                if wb[1 - slot] is not None:
                    wb[1 - slot].wait()
                    wb[1 - slot] = None
                g = issue(1 - slot, ci + 1)
        for w in wb:
            if w is not None:
                w.wait()

    return k(ids_flat, table)


def _sc_gather_bf16(ids_flat, table, chunk):
    n = ids_flat.shape[0]
    d = table.shape[1]
    per_w = n // _NW
    n_chunks = per_w // chunk
    half = d // 2
    groups = half // 16
    mesh = plsc.VectorSubcoreMesh(core_axis_name="c", subcore_axis_name="s")

    @functools.partial(
        pl.kernel,
        mesh=mesh,
        out_type=jax.ShapeDtypeStruct((n, half), jnp.int32),
        scratch_types=[
            pltpu.VMEM((per_w,), jnp.int32),
            pltpu.VMEM((2, chunk, d), jnp.float32),
            pltpu.VMEM((2, chunk, d // 2), jnp.int32),
            pltpu.SemaphoreType.DMA,
            pltpu.SemaphoreType.DMA,
        ],
    )
    def k(ids_hbm, table_hbm, out_hbm, idx_v, rows_v, pk_v, gsem, osem):
        wid = lax.axis_index("s") * _NC + lax.axis_index("c")
        base = wid * per_w
        pltpu.sync_copy(ids_hbm.at[pl.ds(base, per_w)], idx_v)

        def issue(slot, ci):
            return pltpu.async_copy(
                table_hbm.at[idx_v.at[pl.ds(ci * chunk, chunk)]],
                rows_v.at[slot], gsem)

        def pack_chunk(slot):
            src = rows_v.at[slot]
            dst = pk_v.at[slot]

            def row_body(r, _):
                for g in range(groups):
                    a = src[r, pl.ds(16 * g, 16)]
                    b = src[r, pl.ds(half + 16 * g, 16)]
                    ua = lax.bitcast_convert_type(a, jnp.int32)
                    ub = lax.bitcast_convert_type(b, jnp.int32)
                    one = jnp.int32(1)
                    ra = lax.shift_right_logical(
                        ua + jnp.int32(0x7FFF)
                        + (lax.shift_right_logical(ua, 16) & one), 16)
                    rb = lax.shift_right_logical(
                        ub + jnp.int32(0x7FFF)
                        + (lax.shift_right_logical(ub, 16) & one), 16)
                    word = ra | lax.shift_left(rb, 16)
                    dst[r, pl.ds(16 * g, 16)] = word
                return _

            lax.fori_loop(0, chunk, row_body, 0)

        g = issue(0, 0)
        wb = [None, None]
        for ci in range(n_chunks):
            slot = ci % 2
            g.wait()
            if ci + 1 < n_chunks:
                g = issue(1 - slot, ci + 1)
            if wb[slot] is not None:
                wb[slot].wait()
                wb[slot] = None
            pack_chunk(slot)
            off = base + ci * chunk
            wb[slot] = pltpu.async_copy(pk_v.at[slot],
                                        out_hbm.at[pl.ds(off, chunk)], osem)
        for w in wb:
            if w is not None:
                w.wait()

    return k(ids_flat, table)


def _ln_body(w_ref, pos_ref, segf_ref, segtab_ref, gam_ref, bet_ref, o_ref):
    w = w_ref[...]
    lo = lax.bitcast_convert_type(lax.shift_left(w, 16), jnp.float32)
    hi = lax.bitcast_convert_type(w & jnp.int32(-65536), jnp.float32)
    h = lo.shape[1]
    d = 2 * h
    pos = pos_ref[...]
    f = segf_ref[...].astype(jnp.float32)
    s0 = segtab_ref[0:1, :]
    s1 = segtab_ref[1:2, :]
    xl = lo + pos[:, :h] + (s0[:, :h] + f * (s1[:, :h] - s0[:, :h]))
    xh = hi + pos[:, h:] + (s0[:, h:] + f * (s1[:, h:] - s0[:, h:]))
    mean = (jnp.sum(xl, axis=1, keepdims=True)
            + jnp.sum(xh, axis=1, keepdims=True)) / d
    xcl = xl - mean
    xch = xh - mean
    var = (jnp.sum(xcl * xcl, axis=1, keepdims=True)
           + jnp.sum(xch * xch, axis=1, keepdims=True)) / d
    r = lax.rsqrt(var + LN_EPS)
    o_ref[:, :h] = xcl * r * gam_ref[:, :h] + bet_ref[:, :h]
    o_ref[:, h:] = xch * r * gam_ref[:, h:] + bet_ref[:, h:]


def _tc_ln(packed, pos_table, seg_f, segment_table, gamma2d, beta2d,
           block_rows):
    n, hw = packed.shape
    d = 2 * hw
    seq = pos_table.shape[0]
    pos_blocks = seq // block_rows
    batch = n // seq

    return pl.pallas_call(
        _ln_body,
        grid=(pos_blocks, batch),
        in_specs=[
            pl.BlockSpec((block_rows, hw),
                         lambda p, b: (b * pos_blocks + p, 0)),
            pl.BlockSpec((block_rows, d), lambda p, b: (p, 0)),
            pl.BlockSpec((block_rows, 1), lambda p, b: (b * pos_blocks + p, 0)),
            pl.BlockSpec(segment_table.shape, lambda p, b: (0, 0)),
            pl.BlockSpec((1, d), lambda p, b: (0, 0)),
            pl.BlockSpec((1, d), lambda p, b: (0, 0)),
        ],
        out_specs=pl.BlockSpec((block_rows, d),
                               lambda p, b: (b * pos_blocks + p, 0)),
        out_shape=jax.ShapeDtypeStruct((n, d), jnp.float32),
    )(packed, pos_table, seg_f, segment_table, gamma2d, beta2d)


def kernel(input_ids, segment_ids, token_table, segment_table, pos_table,
           ln_gamma, ln_beta):
    batch, seq = input_ids.shape
    d = token_table.shape[1]
    n = batch * seq

    ids_flat = input_ids.reshape(-1).astype(jnp.int32)
    seg_f = segment_ids.reshape(-1, 1).astype(jnp.int32)
    gamma2d = ln_gamma.reshape(1, d)
    beta2d = ln_beta.reshape(1, d)

    packed = _sc_gather_bf16(ids_flat, token_table, chunk=32)
    out = _tc_ln(packed, pos_table, seg_f, segment_table,
                 gamma2d, beta2d, block_rows=2048)
    return out.reshape(batch, seq, d)


# speedup vs baseline: 1.1535x; 1.1535x over previous
import functools

import jax
import jax.numpy as jnp
from jax import lax
from jax.experimental import pallas as pl
from jax.experimental.pallas import tpu as pltpu
from jax.experimental.pallas import tpu_sc as plsc

LN_EPS = 1e-5

_info = plsc.get_sparse_core_info()
_NC, _NS = _info.num_cores, _info.num_subcores
_NW = _NC * _NS


def _sc_gather(ids_flat, table, chunk):
    n = ids_flat.shape[0]
    d = table.shape[1]
    per_w = n // _NW
    n_chunks = per_w // chunk
    mesh = plsc.VectorSubcoreMesh(core_axis_name="c", subcore_axis_name="s")

    @functools.partial(
        pl.kernel,
        mesh=mesh,
        out_type=jax.ShapeDtypeStruct((n, d), jnp.float32),
        scratch_types=[
            pltpu.VMEM((per_w,), jnp.int32),
            pltpu.VMEM((2, chunk, d), jnp.float32),
            pltpu.SemaphoreType.DMA,
            pltpu.SemaphoreType.DMA,
        ],
    )
    def k(ids_hbm, table_hbm, out_hbm, idx_v, rows_v, gsem, osem):
        wid = lax.axis_index("s") * _NC + lax.axis_index("c")
        base = wid * per_w
        pltpu.sync_copy(ids_hbm.at[pl.ds(base, per_w)], idx_v)

        def issue(slot, ci):
            return pltpu.async_copy(
                table_hbm.at[idx_v.at[pl.ds(ci * chunk, chunk)]],
                rows_v.at[slot], gsem)

        g = issue(0, 0)
        wb = [None, None]
        for ci in range(n_chunks):
            slot = ci % 2
            g.wait()
            off = base + ci * chunk
            wb[slot] = pltpu.async_copy(rows_v.at[slot],
                                        out_hbm.at[pl.ds(off, chunk)], osem)
            if ci + 1 < n_chunks:
                if wb[1 - slot] is not None:
                    wb[1 - slot].wait()
                    wb[1 - slot] = None
                g = issue(1 - slot, ci + 1)
        for w in wb:
            if w is not None:
                w.wait()

    return k(ids_flat, table)


def _sc_gather_bf16(ids_flat, table, chunk):
    n = ids_flat.shape[0]
    d = table.shape[1]
    per_w = n // _NW
    n_chunks = per_w // chunk
    half = d // 2
    groups = half // 16
    mesh = plsc.VectorSubcoreMesh(core_axis_name="c", subcore_axis_name="s")

    @functools.partial(
        pl.kernel,
        mesh=mesh,
        out_type=jax.ShapeDtypeStruct((n, half), jnp.int32),
        scratch_types=[
            pltpu.VMEM((per_w,), jnp.int32),
            pltpu.VMEM((2, chunk, d), jnp.float32),
            pltpu.VMEM((2, chunk, d // 2), jnp.int32),
            pltpu.SemaphoreType.DMA,
            pltpu.SemaphoreType.DMA,
        ],
    )
    def k(ids_hbm, table_hbm, out_hbm, idx_v, rows_v, pk_v, gsem, osem):
        wid = lax.axis_index("s") * _NC + lax.axis_index("c")
        base = wid * per_w
        pltpu.sync_copy(ids_hbm.at[pl.ds(base, per_w)], idx_v)

        def issue(slot, ci):
            return pltpu.async_copy(
                table_hbm.at[idx_v.at[pl.ds(ci * chunk, chunk)]],
                rows_v.at[slot], gsem)

        def pack_chunk(slot):
            src = rows_v.at[slot]
            dst = pk_v.at[slot]

            def row_body(i, _):
                for rr in range(2):
                    r = i * 2 + rr
                    for g in range(groups):
                        a = src[r, pl.ds(16 * g, 16)]
                        b = src[r, pl.ds(half + 16 * g, 16)]
                        ua = lax.bitcast_convert_type(a, jnp.int32)
                        ub = lax.bitcast_convert_type(b, jnp.int32)
                        ra = lax.shift_right_logical(
                            ua + jnp.int32(0x8000), 16)
                        rbh = (ub + jnp.int32(0x8000)) & jnp.int32(-65536)
                        dst[r, pl.ds(16 * g, 16)] = ra | rbh
                return _

            lax.fori_loop(0, chunk // 2, row_body, 0)

        g = issue(0, 0)
        wb = [None, None]
        for ci in range(n_chunks):
            slot = ci % 2
            g.wait()
            if ci + 1 < n_chunks:
                g = issue(1 - slot, ci + 1)
            if wb[slot] is not None:
                wb[slot].wait()
                wb[slot] = None
            pack_chunk(slot)
            off = base + ci * chunk
            wb[slot] = pltpu.async_copy(pk_v.at[slot],
                                        out_hbm.at[pl.ds(off, chunk)], osem)
        for w in wb:
            if w is not None:
                w.wait()

    return k(ids_flat, table)


def _ln_body(w_ref, pos_ref, segf_ref, segtab_ref, gam_ref, bet_ref, o_ref):
    w = w_ref[...]
    lo = lax.bitcast_convert_type(lax.shift_left(w, 16), jnp.float32)
    hi = lax.bitcast_convert_type(w & jnp.int32(-65536), jnp.float32)
    h = lo.shape[1]
    d = 2 * h
    pos = pos_ref[...]
    f = segf_ref[...].astype(jnp.float32)
    s0 = segtab_ref[0:1, :]
    s1 = segtab_ref[1:2, :]
    xl = lo + pos[:, :h] + (s0[:, :h] + f * (s1[:, :h] - s0[:, :h]))
    xh = hi + pos[:, h:] + (s0[:, h:] + f * (s1[:, h:] - s0[:, h:]))
    mean = (jnp.sum(xl, axis=1, keepdims=True)
            + jnp.sum(xh, axis=1, keepdims=True)) / d
    xcl = xl - mean
    xch = xh - mean
    var = (jnp.sum(xcl * xcl, axis=1, keepdims=True)
           + jnp.sum(xch * xch, axis=1, keepdims=True)) / d
    r = lax.rsqrt(var + LN_EPS)
    o_ref[:, :h] = xcl * r * gam_ref[:, :h] + bet_ref[:, :h]
    o_ref[:, h:] = xch * r * gam_ref[:, h:] + bet_ref[:, h:]


def _tc_ln(packed, pos_table, seg_f, segment_table, gamma2d, beta2d,
           block_rows):
    n, hw = packed.shape
    d = 2 * hw
    seq = pos_table.shape[0]
    pos_blocks = seq // block_rows
    batch = n // seq

    return pl.pallas_call(
        _ln_body,
        grid=(pos_blocks, batch),
        in_specs=[
            pl.BlockSpec((block_rows, hw),
                         lambda p, b: (b * pos_blocks + p, 0)),
            pl.BlockSpec((block_rows, d), lambda p, b: (p, 0)),
            pl.BlockSpec((block_rows, 1), lambda p, b: (b * pos_blocks + p, 0)),
            pl.BlockSpec(segment_table.shape, lambda p, b: (0, 0)),
            pl.BlockSpec((1, d), lambda p, b: (0, 0)),
            pl.BlockSpec((1, d), lambda p, b: (0, 0)),
        ],
        out_specs=pl.BlockSpec((block_rows, d),
                               lambda p, b: (b * pos_blocks + p, 0)),
        out_shape=jax.ShapeDtypeStruct((n, d), jnp.float32),
    )(packed, pos_table, seg_f, segment_table, gamma2d, beta2d)


def kernel(input_ids, segment_ids, token_table, segment_table, pos_table,
           ln_gamma, ln_beta):
    batch, seq = input_ids.shape
    d = token_table.shape[1]
    n = batch * seq

    ids_flat = input_ids.reshape(-1).astype(jnp.int32)
    seg_f = segment_ids.reshape(-1, 1).astype(jnp.int32)
    gamma2d = ln_gamma.reshape(1, d)
    beta2d = ln_beta.reshape(1, d)

    packed = _sc_gather_bf16(ids_flat, token_table, chunk=32)
    out = _tc_ln(packed, pos_table, seg_f, segment_table,
                 gamma2d, beta2d, block_rows=2048)
    return out.reshape(batch, seq, d)


# speedup vs baseline: 1.5438x; 1.3383x over previous
import functools

import jax
import jax.numpy as jnp
from jax import lax
from jax.experimental import pallas as pl
from jax.experimental.pallas import tpu as pltpu
from jax.experimental.pallas import tpu_sc as plsc

LN_EPS = 1e-5

_info = plsc.get_sparse_core_info()
_NC, _NS = _info.num_cores, _info.num_subcores
_NW = _NC * _NS


def _sc_gather(ids_flat, table, chunk):
    n = ids_flat.shape[0]
    d = table.shape[1]
    per_w = n // _NW
    n_chunks = per_w // chunk
    mesh = plsc.VectorSubcoreMesh(core_axis_name="c", subcore_axis_name="s")

    @functools.partial(
        pl.kernel,
        mesh=mesh,
        out_type=jax.ShapeDtypeStruct((n, d), jnp.float32),
        scratch_types=[
            pltpu.VMEM((per_w,), jnp.int32),
            pltpu.VMEM((2, chunk, d), jnp.float32),
            pltpu.SemaphoreType.DMA,
            pltpu.SemaphoreType.DMA,
        ],
    )
    def k(ids_hbm, table_hbm, out_hbm, idx_v, rows_v, gsem, osem):
        wid = lax.axis_index("s") * _NC + lax.axis_index("c")
        base = wid * per_w
        pltpu.sync_copy(ids_hbm.at[pl.ds(base, per_w)], idx_v)

        def issue(slot, ci):
            return pltpu.async_copy(
                table_hbm.at[idx_v.at[pl.ds(ci * chunk, chunk)]],
                rows_v.at[slot], gsem)

        g = issue(0, 0)
        wb = [None, None]
        for ci in range(n_chunks):
            slot = ci % 2
            g.wait()
            off = base + ci * chunk
            wb[slot] = pltpu.async_copy(rows_v.at[slot],
                                        out_hbm.at[pl.ds(off, chunk)], osem)
            if ci + 1 < n_chunks:
                if wb[1 - slot] is not None:
                    wb[1 - slot].wait()
                    wb[1 - slot] = None
                g = issue(1 - slot, ci + 1)
        for w in wb:
            if w is not None:
                w.wait()

    return k(ids_flat, table)


def _ln_body(g_ref, pos_ref, segf_ref, segtab_ref, gam_ref, bet_ref, o_ref):
    x = g_ref[...] + pos_ref[...]
    s0 = segtab_ref[0:1, :]
    s1 = segtab_ref[1:2, :]
    x = x + s0 + segf_ref[...].astype(jnp.float32) * (s1 - s0)
    mean = jnp.mean(x, axis=1, keepdims=True)
    xc = x - mean
    var = jnp.mean(xc * xc, axis=1, keepdims=True)
    y = xc * lax.rsqrt(var + LN_EPS)
    o_ref[...] = y * gam_ref[...] + bet_ref[...]


def _tc_ln(gathered, pos_table, seg_f, segment_table, gamma2d, beta2d,
           block_rows):
    n, d = gathered.shape
    seq = pos_table.shape[0]
    pos_blocks = seq // block_rows
    batch = n // seq

    return pl.pallas_call(
        _ln_body,
        grid=(pos_blocks, batch),
        in_specs=[
            pl.BlockSpec((block_rows, d), lambda p, b: (b * pos_blocks + p, 0)),
            pl.BlockSpec((block_rows, d), lambda p, b: (p, 0)),
            pl.BlockSpec((block_rows, 1), lambda p, b: (b * pos_blocks + p, 0)),
            pl.BlockSpec(segment_table.shape, lambda p, b: (0, 0)),
            pl.BlockSpec((1, d), lambda p, b: (0, 0)),
            pl.BlockSpec((1, d), lambda p, b: (0, 0)),
        ],
        out_specs=pl.BlockSpec((block_rows, d),
                               lambda p, b: (b * pos_blocks + p, 0)),
        out_shape=jax.ShapeDtypeStruct((n, d), jnp.float32),
    )(gathered, pos_table, seg_f, segment_table, gamma2d, beta2d)


def kernel(input_ids, segment_ids, token_table, segment_table, pos_table,
           ln_gamma, ln_beta):
    batch, seq = input_ids.shape
    d = token_table.shape[1]

    ids_flat = input_ids.reshape(-1).astype(jnp.int32)
    seg_f = segment_ids.reshape(-1, 1).astype(jnp.int32)
    gamma2d = ln_gamma.reshape(1, d)
    beta2d = ln_beta.reshape(1, d)

    gathered = _sc_gather(ids_flat, token_table, chunk=64)
    out = _tc_ln(gathered, pos_table, seg_f, segment_table,
                 gamma2d, beta2d, block_rows=2048)
    return out.reshape(batch, seq, d)


# speedup vs baseline: 1.5618x; 1.0117x over previous
import functools

import jax
import jax.numpy as jnp
from jax import lax
from jax.experimental import pallas as pl
from jax.experimental.pallas import tpu as pltpu
from jax.experimental.pallas import tpu_sc as plsc

LN_EPS = 1e-5

_info = plsc.get_sparse_core_info()
_NC, _NS = _info.num_cores, _info.num_subcores
_NW = _NC * _NS


def _sc_gather(ids_flat, table, chunk):
    n = ids_flat.shape[0]
    d = table.shape[1]
    per_w = n // _NW
    n_chunks = per_w // chunk
    mesh = plsc.VectorSubcoreMesh(core_axis_name="c", subcore_axis_name="s")

    @functools.partial(
        pl.kernel,
        mesh=mesh,
        out_type=jax.ShapeDtypeStruct((n, d), jnp.float32),
        scratch_types=[
            pltpu.VMEM((per_w,), jnp.int32),
            pltpu.VMEM((4, chunk, d), jnp.float32),
            pltpu.SemaphoreType.DMA,
            pltpu.SemaphoreType.DMA,
        ],
    )
    def k(ids_hbm, table_hbm, out_hbm, idx_v, rows_v, gsem, osem):
        nslots = 4
        wid = lax.axis_index("s") * _NC + lax.axis_index("c")
        base = wid * per_w
        pltpu.sync_copy(ids_hbm.at[pl.ds(base, per_w)], idx_v)

        def issue(slot, ci):
            return pltpu.async_copy(
                table_hbm.at[idx_v.at[pl.ds(ci * chunk, chunk)]],
                rows_v.at[slot], gsem)

        ga = [None] * nslots
        wb = [None] * nslots
        for ci in range(min(nslots, n_chunks)):
            ga[ci] = issue(ci, ci)
        for ci in range(n_chunks):
            slot = ci % nslots
            ga[slot].wait()
            off = base + ci * chunk
            wb[slot] = pltpu.async_copy(rows_v.at[slot],
                                        out_hbm.at[pl.ds(off, chunk)], osem)
            nxt = ci + nslots
            if nxt < n_chunks:
                wb[slot].wait()
                wb[slot] = None
                ga[slot] = issue(slot, nxt)
        for w in wb:
            if w is not None:
                w.wait()

    return k(ids_flat, table)


def _ln_body(g_ref, pos_ref, segf_ref, segtab_ref, gam_ref, bet_ref, o_ref):
    x = g_ref[...] + pos_ref[...]
    s0 = segtab_ref[0:1, :]
    s1 = segtab_ref[1:2, :]
    x = x + s0 + segf_ref[...].astype(jnp.float32) * (s1 - s0)
    mean = jnp.mean(x, axis=1, keepdims=True)
    xc = x - mean
    var = jnp.mean(xc * xc, axis=1, keepdims=True)
    y = xc * lax.rsqrt(var + LN_EPS)
    o_ref[...] = y * gam_ref[...] + bet_ref[...]


def _tc_ln(gathered, pos_table, seg_f, segment_table, gamma2d, beta2d,
           block_rows):
    n, d = gathered.shape
    seq = pos_table.shape[0]
    pos_blocks = seq // block_rows
    batch = n // seq

    return pl.pallas_call(
        _ln_body,
        grid=(pos_blocks, batch),
        in_specs=[
            pl.BlockSpec((block_rows, d), lambda p, b: (b * pos_blocks + p, 0)),
            pl.BlockSpec((block_rows, d), lambda p, b: (p, 0)),
            pl.BlockSpec((block_rows, 1), lambda p, b: (b * pos_blocks + p, 0)),
            pl.BlockSpec(segment_table.shape, lambda p, b: (0, 0)),
            pl.BlockSpec((1, d), lambda p, b: (0, 0)),
            pl.BlockSpec((1, d), lambda p, b: (0, 0)),
        ],
        out_specs=pl.BlockSpec((block_rows, d),
                               lambda p, b: (b * pos_blocks + p, 0)),
        out_shape=jax.ShapeDtypeStruct((n, d), jnp.float32),
    )(gathered, pos_table, seg_f, segment_table, gamma2d, beta2d)


def kernel(input_ids, segment_ids, token_table, segment_table, pos_table,
           ln_gamma, ln_beta):
    batch, seq = input_ids.shape
    d = token_table.shape[1]

    ids_flat = input_ids.reshape(-1).astype(jnp.int32)
    seg_f = segment_ids.reshape(-1, 1).astype(jnp.int32)
    gamma2d = ln_gamma.reshape(1, d)
    beta2d = ln_beta.reshape(1, d)

    gathered = _sc_gather(ids_flat, token_table, chunk=32)
    out = _tc_ln(gathered, pos_table, seg_f, segment_table,
                 gamma2d, beta2d, block_rows=2048)
    return out.reshape(batch, seq, d)


# speedup vs baseline: 1.5642x; 1.0015x over previous
import functools

import jax
import jax.numpy as jnp
from jax import lax
from jax.experimental import pallas as pl
from jax.experimental.pallas import tpu as pltpu
from jax.experimental.pallas import tpu_sc as plsc

LN_EPS = 1e-5

_info = plsc.get_sparse_core_info()
_NC, _NS = _info.num_cores, _info.num_subcores
_NW = _NC * _NS


def _sc_gather(ids_flat, table, chunk):
    n = ids_flat.shape[0]
    d = table.shape[1]
    per_w = n // _NW
    n_chunks = per_w // chunk
    mesh = plsc.VectorSubcoreMesh(core_axis_name="c", subcore_axis_name="s")

    @functools.partial(
        pl.kernel,
        mesh=mesh,
        out_type=jax.ShapeDtypeStruct((n, d), jnp.float32),
        scratch_types=[
            pltpu.VMEM((per_w,), jnp.int32),
            pltpu.VMEM((8, chunk, d), jnp.float32),
            pltpu.SemaphoreType.DMA,
            pltpu.SemaphoreType.DMA,
        ],
    )
    def k(ids_hbm, table_hbm, out_hbm, idx_v, rows_v, gsem, osem):
        nslots = 8
        wid = lax.axis_index("s") * _NC + lax.axis_index("c")
        base = wid * per_w
        pltpu.sync_copy(ids_hbm.at[pl.ds(base, per_w)], idx_v)

        def issue(slot, ci):
            return pltpu.async_copy(
                table_hbm.at[idx_v.at[pl.ds(ci * chunk, chunk)]],
                rows_v.at[slot], gsem)

        ga = [None] * nslots
        wb = [None] * nslots
        for ci in range(min(nslots, n_chunks)):
            ga[ci] = issue(ci, ci)
        for ci in range(n_chunks):
            slot = ci % nslots
            ga[slot].wait()
            off = base + ci * chunk
            wb[slot] = pltpu.async_copy(rows_v.at[slot],
                                        out_hbm.at[pl.ds(off, chunk)], osem)
            nxt = ci + nslots
            if nxt < n_chunks:
                wb[slot].wait()
                wb[slot] = None
                ga[slot] = issue(slot, nxt)
        for w in wb:
            if w is not None:
                w.wait()

    return k(ids_flat, table)


def _ln_body(g_ref, pos_ref, segf_ref, segtab_ref, gam_ref, bet_ref, o_ref):
    x = g_ref[...] + pos_ref[...]
    s0 = segtab_ref[0:1, :]
    s1 = segtab_ref[1:2, :]
    x = x + s0 + segf_ref[...].astype(jnp.float32) * (s1 - s0)
    mean = jnp.mean(x, axis=1, keepdims=True)
    xc = x - mean
    var = jnp.mean(xc * xc, axis=1, keepdims=True)
    y = xc * lax.rsqrt(var + LN_EPS)
    o_ref[...] = y * gam_ref[...] + bet_ref[...]


def _tc_ln(gathered, pos_table, seg_f, segment_table, gamma2d, beta2d,
           block_rows):
    n, d = gathered.shape
    seq = pos_table.shape[0]
    pos_blocks = seq // block_rows
    batch = n // seq

    return pl.pallas_call(
        _ln_body,
        grid=(pos_blocks, batch),
        in_specs=[
            pl.BlockSpec((block_rows, d), lambda p, b: (b * pos_blocks + p, 0)),
            pl.BlockSpec((block_rows, d), lambda p, b: (p, 0)),
            pl.BlockSpec((block_rows, 1), lambda p, b: (b * pos_blocks + p, 0)),
            pl.BlockSpec(segment_table.shape, lambda p, b: (0, 0)),
            pl.BlockSpec((1, d), lambda p, b: (0, 0)),
            pl.BlockSpec((1, d), lambda p, b: (0, 0)),
        ],
        out_specs=pl.BlockSpec((block_rows, d),
                               lambda p, b: (b * pos_blocks + p, 0)),
        out_shape=jax.ShapeDtypeStruct((n, d), jnp.float32),
    )(gathered, pos_table, seg_f, segment_table, gamma2d, beta2d)


def kernel(input_ids, segment_ids, token_table, segment_table, pos_table,
           ln_gamma, ln_beta):
    batch, seq = input_ids.shape
    d = token_table.shape[1]

    ids_flat = input_ids.reshape(-1).astype(jnp.int32)
    seg_f = segment_ids.reshape(-1, 1).astype(jnp.int32)
    gamma2d = ln_gamma.reshape(1, d)
    beta2d = ln_beta.reshape(1, d)

    gathered = _sc_gather(ids_flat, token_table, chunk=16)
    out = _tc_ln(gathered, pos_table, seg_f, segment_table,
                 gamma2d, beta2d, block_rows=2048)
    return out.reshape(batch, seq, d)
